# speedup vs baseline: 3.9488x; 3.9488x over previous
import functools

import jax
import jax.numpy as jnp
from jax import lax
from jax.experimental import pallas as pl
from jax.experimental.pallas import tpu as pltpu
from jax.experimental.pallas import tpu_sc as plsc

MAXLEN = 200
EMBED_DIM = 128
BATCH = 1024

NC = 2
NS = 16
LANES = 16
NW = NC * NS
SEQ_PER_W = BATCH // NW
IDX_CHUNK = 100
N_CHUNK = MAXLEN // IDX_CHUNK


def _body(x_hbm, tok_hbm, pos_hbm, out_hbm, idx_v, rows_v, pos_v, sem):
    wid = lax.axis_index("s") * NC + lax.axis_index("c")

    pltpu.sync_copy(pos_hbm, pos_v)

    def seq_step(s, carry):
        b = wid * SEQ_PER_W + s
        pltpu.sync_copy(x_hbm.at[b], idx_v)
        cps = []
        for j in range(N_CHUNK):
            cps.append(
                pltpu.async_copy(
                    tok_hbm.at[idx_v.at[j]],
                    rows_v.at[pl.ds(j * IDX_CHUNK, IDX_CHUNK)],
                    sem,
                )
            )
        for cp in cps:
            cp.wait()

        def row_step(r, c2):
            for c in range(EMBED_DIM // LANES):
                sl = pl.ds(c * LANES, LANES)
                rows_v[r, sl] = rows_v[r, sl] + pos_v[r, sl]
            return c2

        lax.fori_loop(0, MAXLEN, row_step, 0, unroll=False)
        pltpu.sync_copy(rows_v, out_hbm.at[pl.ds(b * MAXLEN, MAXLEN)])
        return carry

    lax.fori_loop(0, SEQ_PER_W, seq_step, 0, unroll=False)


@jax.jit
def _embed(x2, token_table, pos_table):
    mesh = plsc.VectorSubcoreMesh(
        core_axis_name="c", subcore_axis_name="s", num_cores=NC, num_subcores=NS
    )
    run = functools.partial(
        pl.kernel,
        mesh=mesh,
        out_type=jax.ShapeDtypeStruct((BATCH * MAXLEN, EMBED_DIM), jnp.float32),
        scratch_types=[
            pltpu.VMEM((N_CHUNK, IDX_CHUNK), jnp.int32),
            pltpu.VMEM((MAXLEN, EMBED_DIM), jnp.float32),
            pltpu.VMEM((MAXLEN, EMBED_DIM), jnp.float32),
            pltpu.SemaphoreType.DMA,
        ],
    )(_body)
    return run(x2, token_table, pos_table)


def kernel(x, token_table, pos_table):
    x2 = x.astype(jnp.int32).reshape(BATCH, N_CHUNK, IDX_CHUNK)
    out = _embed(x2, token_table, pos_table)
    return out.reshape(BATCH, MAXLEN, EMBED_DIM)


# speedup vs baseline: 4.8236x; 1.2215x over previous
import functools

import jax
import jax.numpy as jnp
from jax import lax
from jax.experimental import pallas as pl
from jax.experimental.pallas import tpu as pltpu
from jax.experimental.pallas import tpu_sc as plsc

MAXLEN = 200
EMBED_DIM = 128
BATCH = 1024

NC = 2
NS = 16
LANES = 16
NW = NC * NS
SEQ_PER_W = BATCH // NW
IDX_CHUNK = 100
N_CHUNK = MAXLEN // IDX_CHUNK


def _body(x_hbm, tok_hbm, pos_hbm, out_hbm,
          idx_v, buf_a, buf_b, pos_v,
          gsem_a, gsem_b, wsem_a, wsem_b):
    wid = lax.axis_index("s") * NC + lax.axis_index("c")

    pltpu.sync_copy(x_hbm.at[wid], idx_v)
    pltpu.sync_copy(pos_hbm, pos_v)

    def issue_gather(s, buf, gsem):
        for j in range(N_CHUNK):
            pltpu.async_copy(
                tok_hbm.at[idx_v.at[s * N_CHUNK + j]],
                buf.at[pl.ds(j * IDX_CHUNK, IDX_CHUNK)],
                gsem,
            )

    def wait_gather(buf, gsem):
        pltpu.make_async_copy(tok_hbm.at[pl.ds(0, MAXLEN)], buf, gsem).wait()

    def add_pos(buf):
        def row_step(r, c2):
            for c in range(EMBED_DIM // LANES):
                sl = pl.ds(c * LANES, LANES)
                plsc.addupdate(buf.at[r, sl], pos_v[r, sl])
            return c2
        lax.fori_loop(0, MAXLEN, row_step, 0, unroll=False)

    def issue_wb(s, buf, wsem):
        b = wid * SEQ_PER_W + s
        pltpu.async_copy(buf, out_hbm.at[pl.ds(b * MAXLEN, MAXLEN)], wsem)

    def wait_wb(buf, wsem):
        pltpu.make_async_copy(buf, out_hbm.at[pl.ds(0, MAXLEN)], wsem).wait()

    issue_gather(0, buf_a, gsem_a)

    def pair_step(u, carry):
        s0 = u * 2
        wait_gather(buf_a, gsem_a)
        add_pos(buf_a)
        issue_wb(s0, buf_a, wsem_a)

        @pl.when(u > 0)
        def _():
            wait_wb(buf_b, wsem_b)
        issue_gather(s0 + 1, buf_b, gsem_b)

        wait_gather(buf_b, gsem_b)
        add_pos(buf_b)
        issue_wb(s0 + 1, buf_b, wsem_b)

        wait_wb(buf_a, wsem_a)

        @pl.when(u < SEQ_PER_W // 2 - 1)
        def _():
            issue_gather(s0 + 2, buf_a, gsem_a)

        return carry

    lax.fori_loop(0, SEQ_PER_W // 2, pair_step, 0, unroll=False)
    wait_wb(buf_b, wsem_b)


@jax.jit
def _embed(x2, token_table, pos_table):
    mesh = plsc.VectorSubcoreMesh(
        core_axis_name="c", subcore_axis_name="s", num_cores=NC, num_subcores=NS
    )
    run = functools.partial(
        pl.kernel,
        mesh=mesh,
        out_type=jax.ShapeDtypeStruct((BATCH * MAXLEN, EMBED_DIM), jnp.float32),
        scratch_types=[
            pltpu.VMEM((SEQ_PER_W * N_CHUNK, IDX_CHUNK), jnp.int32),
            pltpu.VMEM((MAXLEN, EMBED_DIM), jnp.float32),
            pltpu.VMEM((MAXLEN, EMBED_DIM), jnp.float32),
            pltpu.VMEM((MAXLEN, EMBED_DIM), jnp.float32),
            pltpu.SemaphoreType.DMA,
            pltpu.SemaphoreType.DMA,
            pltpu.SemaphoreType.DMA,
            pltpu.SemaphoreType.DMA,
        ],
    )(_body)
    return run(x2, token_table, pos_table)


def kernel(x, token_table, pos_table):
    x2 = x.astype(jnp.int32).reshape(NW, SEQ_PER_W * N_CHUNK, IDX_CHUNK)
    out = _embed(x2, token_table, pos_table)
    return out.reshape(BATCH, MAXLEN, EMBED_DIM)


# speedup vs baseline: 7.2846x; 1.5102x over previous
import functools

import jax
import jax.numpy as jnp
from jax import lax
from jax.experimental import pallas as pl
from jax.experimental.pallas import tpu as pltpu
from jax.experimental.pallas import tpu_sc as plsc

MAXLEN = 200
EMBED_DIM = 128
BATCH = 1024

NC = 2
NS = 16
LANES = 16
NW = NC * NS
SEQ_PER_W = BATCH // NW
IDX_CHUNK = 100
N_CHUNK = MAXLEN // IDX_CHUNK
NBUF = 3
MAIN_TRIPS = 10


def _body(x_hbm, tok_hbm, pos_hbm, out_hbm, idx_v, pos_v, *bufsem):
    bufs = bufsem[:NBUF]
    gsems = bufsem[NBUF:2 * NBUF]
    wsems = bufsem[2 * NBUF:3 * NBUF]
    wid = lax.axis_index("s") * NC + lax.axis_index("c")

    pltpu.sync_copy(x_hbm.at[wid], idx_v)
    pltpu.sync_copy(pos_hbm, pos_v)

    def issue_gather(s, k):
        for j in range(N_CHUNK):
            pltpu.async_copy(
                tok_hbm.at[idx_v.at[s * N_CHUNK + j]],
                bufs[k].at[pl.ds(j * IDX_CHUNK, IDX_CHUNK)],
                gsems[k],
            )

    def wait_gather(k):
        pltpu.make_async_copy(tok_hbm.at[pl.ds(0, MAXLEN)], bufs[k], gsems[k]).wait()

    def add_pos(k):
        buf = bufs[k]
        def row_step(r, c2):
            for cc in range(EMBED_DIM // LANES):
                sl = pl.ds(cc * LANES, LANES)
                plsc.addupdate(buf.at[r, sl], pos_v[r, sl])
            return c2
        lax.fori_loop(0, MAXLEN, row_step, 0, unroll=2)

    def issue_wb(s, k):
        row = wid * (SEQ_PER_W * MAXLEN) + s * MAXLEN
        pltpu.async_copy(bufs[k], out_hbm.at[pl.ds(row, MAXLEN)], wsems[k])

    def wait_wb(k):
        pltpu.make_async_copy(bufs[k], out_hbm.at[pl.ds(0, MAXLEN)], wsems[k]).wait()

    issue_gather(0, 0)

    def step(u, carry):
        for k in range(NBUF):
            s = u * NBUF + k
            wait_gather(k)

            kn = (k + 1) % NBUF
            if k < 2:
                @pl.when(u > 0)
                def _(kn=kn):
                    wait_wb(kn)
            else:
                wait_wb(kn)
            issue_gather(s + 1, kn)

            add_pos(k)
            issue_wb(s, k)
        return carry

    lax.fori_loop(0, MAIN_TRIPS, step, 0, unroll=False)

    s_tail = MAIN_TRIPS * NBUF
    wait_gather(0)
    wait_wb(1)
    issue_gather(s_tail + 1, 1)
    add_pos(0)
    issue_wb(s_tail, 0)

    wait_gather(1)
    add_pos(1)
    issue_wb(s_tail + 1, 1)

    wait_wb(2)
    wait_wb(0)
    wait_wb(1)


@jax.jit
def _embed(x2, token_table, pos_table):
    mesh = plsc.VectorSubcoreMesh(
        core_axis_name="c", subcore_axis_name="s", num_cores=NC, num_subcores=NS
    )
    run = functools.partial(
        pl.kernel,
        mesh=mesh,
        out_type=jax.ShapeDtypeStruct((BATCH * MAXLEN, EMBED_DIM), jnp.float32),
        scratch_types=[
            pltpu.VMEM((SEQ_PER_W * N_CHUNK, IDX_CHUNK), jnp.int32),
            pltpu.VMEM((MAXLEN, EMBED_DIM), jnp.float32),
        ]
        + [pltpu.VMEM((MAXLEN, EMBED_DIM), jnp.float32) for _ in range(NBUF)]
        + [pltpu.SemaphoreType.DMA for _ in range(2 * NBUF)],
    )(_body)
    return run(x2, token_table, pos_table)


def kernel(x, token_table, pos_table):
    x2 = x.astype(jnp.int32).reshape(NW, SEQ_PER_W * N_CHUNK, IDX_CHUNK)
    out = _embed(x2, token_table, pos_table)
    return out.reshape(BATCH, MAXLEN, EMBED_DIM)


# speedup vs baseline: 7.4784x; 1.0266x over previous
import functools

import jax
import jax.numpy as jnp
from jax import lax
from jax.experimental import pallas as pl
from jax.experimental.pallas import tpu as pltpu
from jax.experimental.pallas import tpu_sc as plsc

MAXLEN = 200
EMBED_DIM = 128
BATCH = 1024

NC = 2
NS = 16
LANES = 16
NW = NC * NS
SEQ_PER_W = BATCH // NW
IDX_CHUNK = 100
N_CHUNK = MAXLEN // IDX_CHUNK
NBUF = 3
MAIN_TRIPS = 10


def _body(x_hbm, tok_hbm, pos_hbm, out_hbm, idx_v, pos_v, *bufsem):
    bufs = bufsem[:NBUF]
    gsems = bufsem[NBUF:2 * NBUF]
    wsems = bufsem[2 * NBUF:3 * NBUF]
    psems = bufsem[3 * NBUF:4 * NBUF]
    wid = lax.axis_index("s") * NC + lax.axis_index("c")

    pltpu.sync_copy(x_hbm.at[wid], idx_v)

    @pl.when(lax.axis_index("s") == 0)
    def _():
        pltpu.sync_copy(pos_hbm, pos_v)

    plsc.subcore_barrier()

    def issue_prefill(k):
        pltpu.async_copy(pos_v, bufs[k], psems[k])

    def wait_prefill(k):
        pltpu.make_async_copy(pos_v, bufs[k], psems[k]).wait()

    def issue_gather(s, k):
        for j in range(N_CHUNK):
            pltpu.async_copy(
                tok_hbm.at[idx_v.at[s * N_CHUNK + j]],
                bufs[k].at[pl.ds(j * IDX_CHUNK, IDX_CHUNK)],
                gsems[k],
                add=True,
            )

    def wait_gather(k):
        pltpu.make_async_copy(tok_hbm.at[pl.ds(0, MAXLEN)], bufs[k], gsems[k]).wait()

    def issue_wb(s, k):
        row = wid * (SEQ_PER_W * MAXLEN) + s * MAXLEN
        pltpu.async_copy(bufs[k], out_hbm.at[pl.ds(row, MAXLEN)], wsems[k])

    def wait_wb(k):
        pltpu.make_async_copy(bufs[k], out_hbm.at[pl.ds(0, MAXLEN)], wsems[k]).wait()

    issue_prefill(0)
    wait_prefill(0)
    issue_gather(0, 0)
    issue_prefill(1)

    def step(u, carry):
        for k in range(NBUF):
            s = u * NBUF + k
            kn = (k + 1) % NBUF
            kp = (k + 2) % NBUF
            wait_gather(k)
            issue_wb(s, k)

            wait_prefill(kn)
            issue_gather(s + 1, kn)

            if k == 0:
                @pl.when(u > 0)
                def _(kp=kp):
                    wait_wb(kp)
            else:
                wait_wb(kp)
            issue_prefill(kp)
        return carry

    lax.fori_loop(0, MAIN_TRIPS, step, 0, unroll=False)

    s_tail = MAIN_TRIPS * NBUF
    wait_gather(0)
    issue_wb(s_tail, 0)
    wait_prefill(1)
    issue_gather(s_tail + 1, 1)
    wait_wb(2)

    wait_gather(1)
    issue_wb(s_tail + 1, 1)

    wait_wb(0)
    wait_wb(1)


@jax.jit
def _embed(x2, token_table, pos_table):
    mesh = plsc.VectorSubcoreMesh(
        core_axis_name="c", subcore_axis_name="s", num_cores=NC, num_subcores=NS
    )
    run = functools.partial(
        pl.kernel,
        mesh=mesh,
        out_type=jax.ShapeDtypeStruct((BATCH * MAXLEN, EMBED_DIM), jnp.float32),
        scratch_types=[
            pltpu.VMEM((SEQ_PER_W * N_CHUNK, IDX_CHUNK), jnp.int32),
            pltpu.VMEM_SHARED((MAXLEN, EMBED_DIM), jnp.float32),
        ]
        + [pltpu.VMEM((MAXLEN, EMBED_DIM), jnp.float32) for _ in range(NBUF)]
        + [pltpu.SemaphoreType.DMA for _ in range(3 * NBUF)],
    )(_body)
    return run(x2, token_table, pos_table)


def kernel(x, token_table, pos_table):
    x2 = x.astype(jnp.int32).reshape(NW, SEQ_PER_W * N_CHUNK, IDX_CHUNK)
    out = _embed(x2, token_table, pos_table)
    return out.reshape(BATCH, MAXLEN, EMBED_DIM)


# speedup vs baseline: 7.4836x; 1.0007x over previous
import functools

import jax
import jax.numpy as jnp
from jax import lax
from jax.experimental import pallas as pl
from jax.experimental.pallas import tpu as pltpu
from jax.experimental.pallas import tpu_sc as plsc

MAXLEN = 200
EMBED_DIM = 128
BATCH = 1024

NC = 2
NS = 16
LANES = 16
NW = NC * NS
SEQ_PER_W = BATCH // NW
IDX_CHUNK = 100
N_CHUNK = MAXLEN // IDX_CHUNK
NBUF = 4
MAIN_TRIPS = SEQ_PER_W // NBUF


def _body(x_hbm, tok_hbm, pos_hbm, out_hbm, idx_v, pos_v, *bufsem):
    bufs = bufsem[:NBUF]
    gsems = bufsem[NBUF:2 * NBUF]
    wsems = bufsem[2 * NBUF:3 * NBUF]
    psems = bufsem[3 * NBUF:4 * NBUF]
    wid = lax.axis_index("s") * NC + lax.axis_index("c")

    pltpu.sync_copy(x_hbm.at[wid], idx_v)

    @pl.when(lax.axis_index("s") == 0)
    def _():
        pltpu.sync_copy(pos_hbm, pos_v)

    plsc.subcore_barrier()

    def issue_prefill(k):
        pltpu.async_copy(pos_v, bufs[k], psems[k])

    def wait_prefill(k):
        pltpu.make_async_copy(pos_v, bufs[k], psems[k]).wait()

    def issue_gather(s, k):
        for j in range(N_CHUNK):
            pltpu.async_copy(
                tok_hbm.at[idx_v.at[s * N_CHUNK + j]],
                bufs[k].at[pl.ds(j * IDX_CHUNK, IDX_CHUNK)],
                gsems[k],
                add=True,
            )

    def wait_gather(k):
        pltpu.make_async_copy(tok_hbm.at[pl.ds(0, MAXLEN)], bufs[k], gsems[k]).wait()

    def issue_wb(s, k):
        row = wid * (SEQ_PER_W * MAXLEN) + s * MAXLEN
        pltpu.async_copy(bufs[k], out_hbm.at[pl.ds(row, MAXLEN)], wsems[k])

    def wait_wb(k):
        pltpu.make_async_copy(bufs[k], out_hbm.at[pl.ds(0, MAXLEN)], wsems[k]).wait()

    issue_prefill(0)
    wait_prefill(0)
    issue_gather(0, 0)
    issue_prefill(1)

    def step(u, carry):
        for k in range(NBUF):
            s = u * NBUF + k
            kn = (k + 1) % NBUF
            kp = (k + 2) % NBUF
            wait_gather(k)
            issue_wb(s, k)

            if k == NBUF - 1:
                @pl.when(u < MAIN_TRIPS - 1)
                def _(s=s, kn=kn):
                    wait_prefill(kn)
                    issue_gather(s + 1, kn)
            else:
                wait_prefill(kn)
                issue_gather(s + 1, kn)

            if k < 2:
                @pl.when(u > 0)
                def _(kp=kp):
                    wait_wb(kp)
            else:
                wait_wb(kp)
            if k >= 2:
                @pl.when(u < MAIN_TRIPS - 1)
                def _(kp=kp):
                    issue_prefill(kp)
            else:
                issue_prefill(kp)
        return carry

    lax.fori_loop(0, MAIN_TRIPS, step, 0, unroll=False)

    wait_wb(2)
    wait_wb(3)


@jax.jit
def _embed(x2, token_table, pos_table):
    mesh = plsc.VectorSubcoreMesh(
        core_axis_name="c", subcore_axis_name="s", num_cores=NC, num_subcores=NS
    )
    run = functools.partial(
        pl.kernel,
        mesh=mesh,
        out_type=jax.ShapeDtypeStruct((BATCH * MAXLEN, EMBED_DIM), jnp.float32),
        scratch_types=[
            pltpu.VMEM((SEQ_PER_W * N_CHUNK, IDX_CHUNK), jnp.int32),
            pltpu.VMEM_SHARED((MAXLEN, EMBED_DIM), jnp.float32),
        ]
        + [pltpu.VMEM((MAXLEN, EMBED_DIM), jnp.float32) for _ in range(NBUF)]
        + [pltpu.SemaphoreType.DMA for _ in range(3 * NBUF)],
    )(_body)
    return run(x2, token_table, pos_table)


def kernel(x, token_table, pos_table):
    x2 = x.astype(jnp.int32).reshape(NW, SEQ_PER_W * N_CHUNK, IDX_CHUNK)
    out = _embed(x2, token_table, pos_table)
    return out.reshape(BATCH, MAXLEN, EMBED_DIM)
